# Initial kernel scaffold; baseline (speedup 1.0000x reference)
import functools

import jax
import jax.numpy as jnp
from jax import lax
from jax.experimental import pallas as pl
from jax.experimental.pallas import tpu as pltpu
from jax.experimental.pallas import tpu_sc as plsc

N = 10000
E = 320000
B = 64
DX = 128
DE = 16
DU = 64
DH = 256
DO = 128

NC = 2
NS = 16
NW = NC * NS
EPW = E // NW
CH = 125
NCHUNK = EPW // CH
GK = 8
NGROUP = NCHUNK // GK
ROWS_PT = N // NS


def _sc_scatter(edge_attr3, dst3):
    mesh = plsc.VectorSubcoreMesh(core_axis_name="c", subcore_axis_name="s")

    @functools.partial(
        pl.kernel,
        mesh=mesh,
        out_type=[
            jax.ShapeDtypeStruct((NC, N, DE), jnp.float32),
            jax.ShapeDtypeStruct((NC, N, DE), jnp.float32),
        ],
        scratch_types=[
            pltpu.VMEM((NCHUNK, CH), jnp.int32),
            pltpu.VMEM((2, GK, CH, DE), jnp.float32),
            pltpu.VMEM((CH, DE), jnp.float32),
            pltpu.VMEM((ROWS_PT, DE), jnp.float32),
            pltpu.VMEM_SHARED((N, DE), jnp.float32),
            pltpu.VMEM_SHARED((N, DE), jnp.float32),
            pltpu.SemaphoreType.DMA,
            pltpu.SemaphoreType.DMA,
        ],
    )
    def k(ea_hbm, dst_hbm, osum_hbm, ocnt_hbm,
          idx_v, vbuf, ones_v, stage_v, acc_s, cnt_s, sem_in, sem_sc):
        c = lax.axis_index("c")
        s = lax.axis_index("s")
        wid = c * NS + s

        def zrow(i, carry):
            stage_v[i, :] = jnp.zeros((DE,), jnp.float32)
            return carry
        lax.fori_loop(0, ROWS_PT, zrow, 0)
        rbase = s * ROWS_PT
        pltpu.sync_copy(stage_v, acc_s.at[pl.ds(rbase, ROWS_PT)])
        pltpu.sync_copy(stage_v, cnt_s.at[pl.ds(rbase, ROWS_PT)])

        def orow(i, carry):
            ones_v[i, :] = jnp.ones((DE,), jnp.float32)
            return carry
        lax.fori_loop(0, CH, orow, 0)

        pltpu.sync_copy(dst_hbm.at[wid], idx_v)
        plsc.subcore_barrier()

        cb0 = wid * NCHUNK
        pltpu.async_copy(ea_hbm.at[pl.ds(cb0, GK)], vbuf.at[0], sem_in)

        def group(g, carry):
            p = lax.rem(g, 2)
            pltpu.make_async_copy(
                ea_hbm.at[pl.ds(cb0 + g * GK, GK)], vbuf.at[p], sem_in).wait()

            @pl.when(g + 1 < NGROUP)
            def _():
                pltpu.async_copy(
                    ea_hbm.at[pl.ds(cb0 + (g + 1) * GK, GK)],
                    vbuf.at[lax.rem(g + 1, 2)], sem_in)

            for kk in range(GK):
                j = g * GK + kk
                pltpu.async_copy(
                    vbuf.at[p, kk], acc_s.at[idx_v.at[j]], sem_sc, add=True)
                pltpu.async_copy(
                    ones_v, cnt_s.at[idx_v.at[j]], sem_sc, add=True)
            for kk in range(GK):
                j = g * GK + kk
                pltpu.make_async_copy(
                    vbuf.at[p, kk], acc_s.at[idx_v.at[j]], sem_sc).wait()
                pltpu.make_async_copy(
                    ones_v, cnt_s.at[idx_v.at[j]], sem_sc).wait()
            return carry
        lax.fori_loop(0, NGROUP, group, 0)

        plsc.subcore_barrier()

        pltpu.sync_copy(acc_s.at[pl.ds(rbase, ROWS_PT)], stage_v)
        pltpu.sync_copy(stage_v, osum_hbm.at[c].at[pl.ds(rbase, ROWS_PT)])
        pltpu.sync_copy(cnt_s.at[pl.ds(rbase, ROWS_PT)], stage_v)
        pltpu.sync_copy(stage_v, ocnt_hbm.at[c].at[pl.ds(rbase, ROWS_PT)])

    return k(edge_attr3, dst3)


def _elu(v):
    return jnp.where(v > 0, v, jnp.expm1(jnp.minimum(v, 0.0)))


def _tc_body(batch_ref, x_ref, u_ref, esum_ref, ecnt_ref,
             wi_ref, bi_ref, w0_ref, b0_ref, w1_ref, b1_ref,
             wo_ref, bnw_ref, bnb_ref, out_ref):
    f32 = jnp.float32
    bcol = batch_ref[...]
    ids = lax.broadcasted_iota(jnp.int32, (N, B), 1)
    mask = (bcol == ids).astype(f32)

    onescol = jnp.full((N, 1), 1.0, f32)
    dn = (((0,), (0,)), ((), ()))
    cnt = lax.dot_general(mask, onescol, dn,
                          preferred_element_type=f32)
    cmax = jnp.maximum(cnt, 1.0)

    nsum = lax.dot_general(mask, x_ref[...], dn,
                           preferred_element_type=f32)
    node_mean = nsum / cmax

    es = esum_ref[0] + esum_ref[1]
    ec = ecnt_ref[0] + ecnt_ref[1]
    emn = es / jnp.maximum(ec, 1.0)
    egs = lax.dot_general(mask, emn, dn,
                          preferred_element_type=f32)
    edge_mean = egs / cmax

    wi = wi_ref[...]
    h = (jnp.dot(u_ref[...], wi[0:DU, :], preferred_element_type=f32)
         + jnp.dot(node_mean, wi[DU:DU + DX, :], preferred_element_type=f32)
         + jnp.dot(edge_mean, wi[DU + DX:, :], preferred_element_type=f32)
         + bi_ref[...])
    h = _elu(h)
    skip = h
    h = _elu(jnp.dot(h, w0_ref[...], preferred_element_type=f32) + b0_ref[...])
    h = _elu(jnp.dot(h, w1_ref[...], preferred_element_type=f32) + b1_ref[...])
    h = h + skip
    y = jnp.dot(h, wo_ref[...], preferred_element_type=f32)
    y = y * (bnw_ref[...] * (1.0 / jnp.sqrt(1.0 + 1e-5))) + bnb_ref[...]
    out_ref[...] = _elu(y)


def kernel(x, edge_index, edge_attr, u, batch, W_in, b_in, W_h0, b_h0,
           W_h1, b_h1, W_out, bn_w, bn_b):
    dst3 = edge_index[1].reshape(NW, NCHUNK, CH)
    ea3 = edge_attr.reshape(E // CH, CH, DE)
    esum, ecnt = _sc_scatter(ea3, dst3)

    out = pl.pallas_call(
        _tc_body,
        out_shape=jax.ShapeDtypeStruct((B, DO), jnp.float32),
    )(batch.reshape(N, 1), x, u, esum, ecnt,
      W_in, b_in.reshape(1, DH), W_h0, b_h0.reshape(1, DH),
      W_h1, b_h1.reshape(1, DH), W_out, bn_w.reshape(1, DO),
      bn_b.reshape(1, DO))
    return out


# speedup vs baseline: 6.3576x; 6.3576x over previous
import functools

import jax
import jax.numpy as jnp
from jax import lax
from jax.experimental import pallas as pl
from jax.experimental.pallas import tpu as pltpu
from jax.experimental.pallas import tpu_sc as plsc

N = 10000
E = 320000
B = 64
DX = 128
DE = 16
DU = 64
DH = 256
DO = 128

NC = 2
NS = 16
NW = NC * NS
EPW = E // NW
CH = 125
NCHUNK = EPW // CH
GK = 8
NGROUP = NCHUNK // GK
ROWS_PT = 624
TAIL = N - ROWS_PT * NS


def _sc_scatter(edge_attr3, dst3):
    mesh = plsc.VectorSubcoreMesh(core_axis_name="c", subcore_axis_name="s")

    @functools.partial(
        pl.kernel,
        mesh=mesh,
        out_type=[
            jax.ShapeDtypeStruct((NC, N, DE), jnp.float32),
            jax.ShapeDtypeStruct((NC, N, DE), jnp.float32),
        ],
        scratch_types=[
            pltpu.VMEM((NCHUNK, CH), jnp.int32),
            pltpu.VMEM((2, GK, CH, DE), jnp.float32),
            pltpu.VMEM((CH, DE), jnp.float32),
            pltpu.VMEM((ROWS_PT, DE), jnp.float32),
            pltpu.VMEM_SHARED((N, DE), jnp.float32),
            pltpu.VMEM_SHARED((N, DE), jnp.float32),
            pltpu.SemaphoreType.DMA,
            pltpu.SemaphoreType.DMA,
        ],
        compiler_params=pltpu.CompilerParams(use_tc_tiling_on_sc=False),
    )
    def k(ea_hbm, dst_hbm, osum_hbm, ocnt_hbm,
          idx_v, vbuf, ones_v, stage_v, acc_s, cnt_s, sem_in, sem_sc):
        c = lax.axis_index("c")
        s = lax.axis_index("s")
        wid = c * NS + s

        def zrow(i, carry):
            stage_v[i, :] = jnp.zeros((DE,), jnp.float32)
            return carry
        lax.fori_loop(0, ROWS_PT, zrow, 0)
        rbase = s * ROWS_PT
        pltpu.sync_copy(stage_v, acc_s.at[pl.ds(rbase, ROWS_PT)])
        pltpu.sync_copy(stage_v, cnt_s.at[pl.ds(rbase, ROWS_PT)])

        @pl.when(s == 0)
        def _():
            tb = NS * ROWS_PT
            pltpu.sync_copy(stage_v.at[pl.ds(0, TAIL)],
                            acc_s.at[pl.ds(tb, TAIL)])
            pltpu.sync_copy(stage_v.at[pl.ds(0, TAIL)],
                            cnt_s.at[pl.ds(tb, TAIL)])

        def orow(i, carry):
            ones_v[i, :] = jnp.ones((DE,), jnp.float32)
            return carry
        lax.fori_loop(0, CH, orow, 0)

        pltpu.sync_copy(dst_hbm.at[wid], idx_v)
        plsc.subcore_barrier()

        cb0 = wid * NCHUNK
        pltpu.async_copy(ea_hbm.at[pl.ds(cb0, GK)], vbuf.at[0], sem_in)

        def group(g, carry):
            p = lax.rem(g, 2)
            pltpu.make_async_copy(
                ea_hbm.at[pl.ds(cb0 + g * GK, GK)], vbuf.at[p], sem_in).wait()

            @pl.when(g + 1 < NGROUP)
            def _():
                pltpu.async_copy(
                    ea_hbm.at[pl.ds(cb0 + (g + 1) * GK, GK)],
                    vbuf.at[lax.rem(g + 1, 2)], sem_in)

            for kk in range(GK):
                j = g * GK + kk
                pltpu.async_copy(
                    vbuf.at[p, kk], acc_s.at[idx_v.at[j]], sem_sc, add=True)
                pltpu.async_copy(
                    ones_v, cnt_s.at[idx_v.at[j]], sem_sc, add=True)
            for kk in range(GK):
                j = g * GK + kk
                pltpu.make_async_copy(
                    vbuf.at[p, kk], acc_s.at[idx_v.at[j]], sem_sc).wait()
                pltpu.make_async_copy(
                    ones_v, cnt_s.at[idx_v.at[j]], sem_sc).wait()
            return carry
        lax.fori_loop(0, NGROUP, group, 0)

        plsc.subcore_barrier()

        pltpu.sync_copy(acc_s.at[pl.ds(rbase, ROWS_PT)], stage_v)
        pltpu.sync_copy(stage_v, osum_hbm.at[c].at[pl.ds(rbase, ROWS_PT)])
        pltpu.sync_copy(cnt_s.at[pl.ds(rbase, ROWS_PT)], stage_v)
        pltpu.sync_copy(stage_v, ocnt_hbm.at[c].at[pl.ds(rbase, ROWS_PT)])

        @pl.when(s == 0)
        def _():
            tb = NS * ROWS_PT
            pltpu.sync_copy(acc_s.at[pl.ds(tb, TAIL)],
                            stage_v.at[pl.ds(0, TAIL)])
            pltpu.sync_copy(stage_v.at[pl.ds(0, TAIL)],
                            osum_hbm.at[c].at[pl.ds(tb, TAIL)])
            pltpu.sync_copy(cnt_s.at[pl.ds(tb, TAIL)],
                            stage_v.at[pl.ds(0, TAIL)])
            pltpu.sync_copy(stage_v.at[pl.ds(0, TAIL)],
                            ocnt_hbm.at[c].at[pl.ds(tb, TAIL)])

    return k(edge_attr3, dst3)


def _elu(v):
    return jnp.where(v > 0, v, jnp.exp(jnp.minimum(v, 0.0)) - 1.0)


def _tc_body(batch_ref, x_ref, u_ref, esum_ref, ecnt_ref,
             wi_ref, bi_ref, w0_ref, b0_ref, w1_ref, b1_ref,
             wo_ref, bnw_ref, bnb_ref, out_ref):
    f32 = jnp.float32
    bcol = batch_ref[...]
    ids = lax.broadcasted_iota(jnp.int32, (N, B), 1)
    mask = (bcol == ids).astype(f32)

    onescol = jnp.full((N, 1), 1.0, f32)
    dn = (((0,), (0,)), ((), ()))
    cnt = lax.dot_general(mask, onescol, dn,
                          preferred_element_type=f32)
    cmax = jnp.maximum(cnt, 1.0)

    nsum = lax.dot_general(mask, x_ref[...], dn,
                           preferred_element_type=f32)
    node_mean = nsum / cmax

    es = esum_ref[0] + esum_ref[1]
    ec = ecnt_ref[0] + ecnt_ref[1]
    emn = es / jnp.maximum(ec, 1.0)
    egs = lax.dot_general(mask, emn, dn,
                          preferred_element_type=f32)
    edge_mean = egs / cmax

    wi = wi_ref[...]
    h = (jnp.dot(u_ref[...], wi[0:DU, :], preferred_element_type=f32)
         + jnp.dot(node_mean, wi[DU:DU + DX, :], preferred_element_type=f32)
         + jnp.dot(edge_mean, wi[DU + DX:, :], preferred_element_type=f32)
         + bi_ref[...])
    h = _elu(h)
    skip = h
    h = _elu(jnp.dot(h, w0_ref[...], preferred_element_type=f32) + b0_ref[...])
    h = _elu(jnp.dot(h, w1_ref[...], preferred_element_type=f32) + b1_ref[...])
    h = h + skip
    y = jnp.dot(h, wo_ref[...], preferred_element_type=f32)
    y = y * (bnw_ref[...] * (1.0 / jnp.sqrt(1.0 + 1e-5))) + bnb_ref[...]
    out_ref[...] = _elu(y)


def kernel(x, edge_index, edge_attr, u, batch, W_in, b_in, W_h0, b_h0,
           W_h1, b_h1, W_out, bn_w, bn_b):
    dst3 = edge_index[1].reshape(NW, NCHUNK, CH)
    ea3 = edge_attr.reshape(E // CH, CH, DE)
    esum, ecnt = _sc_scatter(ea3, dst3)

    out = pl.pallas_call(
        _tc_body,
        out_shape=jax.ShapeDtypeStruct((B, DO), jnp.float32),
    )(batch.reshape(N, 1), x, u, esum, ecnt,
      W_in, b_in.reshape(1, DH), W_h0, b_h0.reshape(1, DH),
      W_h1, b_h1.reshape(1, DH), W_out, bn_w.reshape(1, DO),
      bn_b.reshape(1, DO))
    return out


# speedup vs baseline: 7.2301x; 1.1372x over previous
import functools

import jax
import jax.numpy as jnp
from jax import lax
from jax.experimental import pallas as pl
from jax.experimental.pallas import tpu as pltpu
from jax.experimental.pallas import tpu_sc as plsc

N = 10000
E = 320000
B = 64
DX = 128
DE = 16
DU = 64
DH = 256
DO = 128

NC = 2
NS = 16
NW = NC * NS
EPW = E // NW
CH = 125
NCHUNK = EPW // CH
GK = 8
NGROUP = NCHUNK // GK
ROWS_PT = 624
TAIL = N - ROWS_PT * NS


def _sc_scatter(edge_attr, dst3):
    mesh = plsc.VectorSubcoreMesh(core_axis_name="c", subcore_axis_name="s")

    @functools.partial(
        pl.kernel,
        mesh=mesh,
        out_type=[
            jax.ShapeDtypeStruct((NC, N, DE), jnp.float32),
            jax.ShapeDtypeStruct((NC, N, DE), jnp.float32),
        ],
        scratch_types=[
            pltpu.VMEM((NCHUNK, CH), jnp.int32),
            pltpu.VMEM((2, GK * CH, DE), jnp.float32),
            pltpu.VMEM((CH, DE), jnp.float32),
            pltpu.VMEM((ROWS_PT, DE), jnp.float32),
            pltpu.VMEM_SHARED((N, DE), jnp.float32),
            pltpu.VMEM_SHARED((N, DE), jnp.float32),
            pltpu.SemaphoreType.DMA,
            pltpu.SemaphoreType.DMA,
        ],
        compiler_params=pltpu.CompilerParams(use_tc_tiling_on_sc=False),
    )
    def k(ea_hbm, dst_hbm, osum_hbm, ocnt_hbm,
          idx_v, vbuf, ones_v, stage_v, acc_s, cnt_s, sem_in, sem_sc):
        c = lax.axis_index("c")
        s = lax.axis_index("s")
        wid = c * NS + s

        def zrow(i, carry):
            stage_v[i, :] = jnp.zeros((DE,), jnp.float32)
            return carry
        lax.fori_loop(0, ROWS_PT, zrow, 0)
        rbase = s * ROWS_PT
        pltpu.sync_copy(stage_v, acc_s.at[pl.ds(rbase, ROWS_PT)])
        pltpu.sync_copy(stage_v, cnt_s.at[pl.ds(rbase, ROWS_PT)])

        @pl.when(s == 0)
        def _():
            tb = NS * ROWS_PT
            pltpu.sync_copy(stage_v.at[pl.ds(0, TAIL)],
                            acc_s.at[pl.ds(tb, TAIL)])
            pltpu.sync_copy(stage_v.at[pl.ds(0, TAIL)],
                            cnt_s.at[pl.ds(tb, TAIL)])

        def orow(i, carry):
            ones_v[i, :] = jnp.ones((DE,), jnp.float32)
            return carry
        lax.fori_loop(0, CH, orow, 0)

        pltpu.sync_copy(dst_hbm.at[wid], idx_v)
        plsc.subcore_barrier()

        GR = GK * CH
        eb0 = wid * EPW
        pltpu.async_copy(ea_hbm.at[pl.ds(eb0, GR)], vbuf.at[0], sem_in)

        def group(g, carry):
            p = lax.rem(g, 2)
            pltpu.make_async_copy(
                ea_hbm.at[pl.ds(eb0 + g * GR, GR)], vbuf.at[p], sem_in).wait()

            @pl.when(g + 1 < NGROUP)
            def _():
                pltpu.async_copy(
                    ea_hbm.at[pl.ds(eb0 + (g + 1) * GR, GR)],
                    vbuf.at[lax.rem(g + 1, 2)], sem_in)

            for kk in range(GK):
                j = g * GK + kk
                vals = vbuf.at[p].at[pl.ds(kk * CH, CH)]
                pltpu.async_copy(
                    vals, acc_s.at[idx_v.at[j]], sem_sc, add=True)
                pltpu.async_copy(
                    ones_v, cnt_s.at[idx_v.at[j]], sem_sc, add=True)
            for kk in range(GK):
                j = g * GK + kk
                vals = vbuf.at[p].at[pl.ds(kk * CH, CH)]
                pltpu.make_async_copy(
                    vals, acc_s.at[idx_v.at[j]], sem_sc).wait()
                pltpu.make_async_copy(
                    ones_v, cnt_s.at[idx_v.at[j]], sem_sc).wait()
            return carry
        lax.fori_loop(0, NGROUP, group, 0)

        plsc.subcore_barrier()

        pltpu.sync_copy(acc_s.at[pl.ds(rbase, ROWS_PT)], stage_v)
        pltpu.sync_copy(stage_v, osum_hbm.at[c].at[pl.ds(rbase, ROWS_PT)])
        pltpu.sync_copy(cnt_s.at[pl.ds(rbase, ROWS_PT)], stage_v)
        pltpu.sync_copy(stage_v, ocnt_hbm.at[c].at[pl.ds(rbase, ROWS_PT)])

        @pl.when(s == 0)
        def _():
            tb = NS * ROWS_PT
            pltpu.sync_copy(acc_s.at[pl.ds(tb, TAIL)],
                            stage_v.at[pl.ds(0, TAIL)])
            pltpu.sync_copy(stage_v.at[pl.ds(0, TAIL)],
                            osum_hbm.at[c].at[pl.ds(tb, TAIL)])
            pltpu.sync_copy(cnt_s.at[pl.ds(tb, TAIL)],
                            stage_v.at[pl.ds(0, TAIL)])
            pltpu.sync_copy(stage_v.at[pl.ds(0, TAIL)],
                            ocnt_hbm.at[c].at[pl.ds(tb, TAIL)])

    return k(edge_attr, dst3)


def _elu(v):
    return jnp.where(v > 0, v, jnp.exp(jnp.minimum(v, 0.0)) - 1.0)


def _tc_body(batch_ref, x_ref, u_ref, esum_ref, ecnt_ref,
             wi_ref, bi_ref, w0_ref, b0_ref, w1_ref, b1_ref,
             wo_ref, bnw_ref, bnb_ref, out_ref):
    f32 = jnp.float32
    bcol = batch_ref[...]
    ids = lax.broadcasted_iota(jnp.int32, (N, B), 1)
    mask = (bcol == ids).astype(f32)

    onescol = jnp.full((N, 1), 1.0, f32)
    dn = (((0,), (0,)), ((), ()))
    cnt = lax.dot_general(mask, onescol, dn,
                          preferred_element_type=f32)
    cmax = jnp.maximum(cnt, 1.0)

    nsum = lax.dot_general(mask, x_ref[...], dn,
                           preferred_element_type=f32)
    node_mean = nsum / cmax

    es = esum_ref[0] + esum_ref[1]
    ec = ecnt_ref[0] + ecnt_ref[1]
    emn = es / jnp.maximum(ec, 1.0)
    egs = lax.dot_general(mask, emn, dn,
                          preferred_element_type=f32)
    edge_mean = egs / cmax

    wi = wi_ref[...]
    h = (jnp.dot(u_ref[...], wi[0:DU, :], preferred_element_type=f32)
         + jnp.dot(node_mean, wi[DU:DU + DX, :], preferred_element_type=f32)
         + jnp.dot(edge_mean, wi[DU + DX:, :], preferred_element_type=f32)
         + bi_ref[...])
    h = _elu(h)
    skip = h
    h = _elu(jnp.dot(h, w0_ref[...], preferred_element_type=f32) + b0_ref[...])
    h = _elu(jnp.dot(h, w1_ref[...], preferred_element_type=f32) + b1_ref[...])
    h = h + skip
    y = jnp.dot(h, wo_ref[...], preferred_element_type=f32)
    y = y * (bnw_ref[...] * (1.0 / jnp.sqrt(1.0 + 1e-5))) + bnb_ref[...]
    out_ref[...] = _elu(y)


def kernel(x, edge_index, edge_attr, u, batch, W_in, b_in, W_h0, b_h0,
           W_h1, b_h1, W_out, bn_w, bn_b):
    dst3 = edge_index[1].reshape(NW, NCHUNK, CH)
    esum, ecnt = _sc_scatter(edge_attr, dst3)

    out = pl.pallas_call(
        _tc_body,
        out_shape=jax.ShapeDtypeStruct((B, DO), jnp.float32),
    )(batch.reshape(N, 1), x, u, esum, ecnt,
      W_in, b_in.reshape(1, DH), W_h0, b_h0.reshape(1, DH),
      W_h1, b_h1.reshape(1, DH), W_out, bn_w.reshape(1, DO),
      bn_b.reshape(1, DO))
    return out


# speedup vs baseline: 18.4358x; 2.5499x over previous
import functools

import jax
import jax.numpy as jnp
from jax import lax
from jax.experimental import pallas as pl
from jax.experimental.pallas import tpu as pltpu
from jax.experimental.pallas import tpu_sc as plsc

N = 10000
E = 320000
B = 64
DX = 128
DE = 16
DU = 64
DH = 256
DO = 128

NC = 2
NS = 16
NW = NC * NS
EPW = E // NW
CH = 128
NCHT = E // CH
CPW = NCHT // NW
NTAIL = NCHT - CPW * NW
GK = 6
NGROUP = CPW // GK
GR = GK * CH
TG = GR // 16
ROWS_PT = 624
TAIL = N - ROWS_PT * NS
NPAD = 48
NP = (N + NPAD) * DE // 128


def _sc_scatter(ea4, ei3):
    mesh = plsc.VectorSubcoreMesh(core_axis_name="c", subcore_axis_name="s")

    @functools.partial(
        pl.kernel,
        mesh=mesh,
        out_type=[
            jax.ShapeDtypeStruct((NC, N + NPAD, DE), jnp.float32),
            jax.ShapeDtypeStruct((NC, N + NPAD, DE), jnp.float32),
        ],
        scratch_types=[
            pltpu.VMEM((CPW + 1, CH), jnp.int32),
            pltpu.VMEM((2, GK, 8, CH), jnp.float32),
            pltpu.VMEM((2, GK, 8, CH), jnp.float32),
            pltpu.VMEM((GR, DE), jnp.float32),
            pltpu.VMEM((GR, DE), jnp.float32),
            pltpu.VMEM((CH, DE), jnp.float32),
            pltpu.VMEM((ROWS_PT, DE), jnp.float32),
            pltpu.VMEM_SHARED((N, DE), jnp.float32),
            pltpu.VMEM_SHARED((N, DE), jnp.float32),
            pltpu.SemaphoreType.DMA,
            pltpu.SemaphoreType.DMA,
            pltpu.SemaphoreType.DMA,
            pltpu.SemaphoreType.DMA,
        ],
        compiler_params=pltpu.CompilerParams(
            use_tc_tiling_on_sc=False, needs_layout_passes=False),
    )
    def k(ea_hbm, ei_hbm, osum_hbm, ocnt_hbm,
          idx_v, sbuf0, sbuf1, tbuf0, tbuf1, ones_v, stage_v,
          acc_s, cnt_s, sem_in0, sem_in1, sem_sc0, sem_sc1):
        c = lax.axis_index("c")
        s = lax.axis_index("s")
        wid = c * NS + s
        zeros16 = jnp.zeros((16,), jnp.float32)

        def zrow(i, carry):
            stage_v[i, :] = zeros16
            return carry
        lax.fori_loop(0, ROWS_PT, zrow, 0)
        rbase = s * ROWS_PT
        pltpu.sync_copy(stage_v, acc_s.at[pl.ds(rbase, ROWS_PT)])
        pltpu.sync_copy(stage_v, cnt_s.at[pl.ds(rbase, ROWS_PT)])

        @pl.when(s == 0)
        def _():
            tb = NS * ROWS_PT
            pltpu.sync_copy(stage_v.at[pl.ds(0, TAIL)],
                            acc_s.at[pl.ds(tb, TAIL)])
            pltpu.sync_copy(stage_v.at[pl.ds(0, TAIL)],
                            cnt_s.at[pl.ds(tb, TAIL)])
            pltpu.sync_copy(stage_v.at[pl.ds(0, NPAD)],
                            osum_hbm.at[c].at[pl.ds(N, NPAD)])
            pltpu.sync_copy(stage_v.at[pl.ds(0, NPAD)],
                            ocnt_hbm.at[c].at[pl.ds(N, NPAD)])

        def orow(i, carry):
            ones_v[i, :] = jnp.ones((DE,), jnp.float32)
            return carry
        lax.fori_loop(0, CH, orow, 0)

        c0 = wid * CPW
        pltpu.sync_copy(ei_hbm.at[pl.ds(c0, CPW), 1],
                        idx_v.at[pl.ds(0, CPW)])

        @pl.when(wid < NTAIL)
        def _():
            pltpu.sync_copy(ei_hbm.at[NW * CPW + wid, 1], idx_v.at[CPW])
        plsc.subcore_barrier()

        lane = lax.iota(jnp.int32, 16)
        pltpu.async_copy(ea_hbm.at[:, pl.ds(c0, GK)], sbuf0, sem_in0)

        def drain_group(tbuf, sem):
            for _ in range(GK):
                pltpu.make_async_copy(
                    tbuf.at[pl.ds(0, CH)], acc_s.at[idx_v.at[0]], sem).wait()
                pltpu.make_async_copy(
                    ones_v, cnt_s.at[idx_v.at[0]], sem).wait()

        def transpose_block(sbuf, tbuf, nstep):
            def tstep(t, carry):
                kk = t // 8
                tt = lax.rem(t, 8)
                rowi = t * 16 + lane
                for tr in range(2):
                    for sl in range(8):
                        v = sbuf[tr, kk, sl, pl.ds(tt * 16, 16)]
                        plsc.store_scatter(
                            tbuf,
                            [rowi, jnp.full((16,), tr * 8 + sl, jnp.int32)],
                            v)
                return carry
            lax.fori_loop(0, nstep, tstep, 0)

        def do_group(g, sbuf, tbuf, sem_in, sem_sc):
            pltpu.make_async_copy(
                ea_hbm.at[:, pl.ds(c0 + g * GK, GK)], sbuf, sem_in).wait()

            @pl.when(g >= 2)
            def _():
                drain_group(tbuf, sem_sc)

            transpose_block(sbuf, tbuf, TG)

            for kk in range(GK):
                j = g * GK + kk
                pltpu.async_copy(tbuf.at[pl.ds(kk * CH, CH)],
                                 acc_s.at[idx_v.at[j]], sem_sc, add=True)
                pltpu.async_copy(ones_v, cnt_s.at[idx_v.at[j]],
                                 sem_sc, add=True)

        def group(g, carry):
            @pl.when(g + 1 < NGROUP)
            def _():
                @pl.when(lax.rem(g + 1, 2) == 0)
                def _():
                    pltpu.async_copy(
                        ea_hbm.at[:, pl.ds(c0 + (g + 1) * GK, GK)],
                        sbuf0, sem_in0)

                @pl.when(lax.rem(g + 1, 2) == 1)
                def _():
                    pltpu.async_copy(
                        ea_hbm.at[:, pl.ds(c0 + (g + 1) * GK, GK)],
                        sbuf1, sem_in1)

            @pl.when(lax.rem(g, 2) == 0)
            def _():
                do_group(g, sbuf0, tbuf0, sem_in0, sem_sc0)

            @pl.when(lax.rem(g, 2) == 1)
            def _():
                do_group(g, sbuf1, tbuf1, sem_in1, sem_sc1)
            return carry
        lax.fori_loop(0, NGROUP, group, 0)

        drain_group(tbuf0, sem_sc0)
        drain_group(tbuf1, sem_sc1)

        @pl.when(wid < NTAIL)
        def _():
            pltpu.sync_copy(ea_hbm.at[:, pl.ds(NW * CPW + wid, 1)],
                            sbuf0.at[:, pl.ds(0, 1)])
            transpose_block(sbuf0, tbuf0, CH // 16)
            pltpu.sync_copy(tbuf0.at[pl.ds(0, CH)],
                            acc_s.at[idx_v.at[CPW]], add=True)
            pltpu.sync_copy(ones_v, cnt_s.at[idx_v.at[CPW]], add=True)

        plsc.subcore_barrier()

        pltpu.sync_copy(acc_s.at[pl.ds(rbase, ROWS_PT)], stage_v)
        pltpu.sync_copy(stage_v, osum_hbm.at[c].at[pl.ds(rbase, ROWS_PT)])
        pltpu.sync_copy(cnt_s.at[pl.ds(rbase, ROWS_PT)], stage_v)
        pltpu.sync_copy(stage_v, ocnt_hbm.at[c].at[pl.ds(rbase, ROWS_PT)])

        @pl.when(s == 0)
        def _():
            tb = NS * ROWS_PT
            pltpu.sync_copy(acc_s.at[pl.ds(tb, TAIL)],
                            stage_v.at[pl.ds(0, TAIL)])
            pltpu.sync_copy(stage_v.at[pl.ds(0, TAIL)],
                            osum_hbm.at[c].at[pl.ds(tb, TAIL)])
            pltpu.sync_copy(cnt_s.at[pl.ds(tb, TAIL)],
                            stage_v.at[pl.ds(0, TAIL)])
            pltpu.sync_copy(stage_v.at[pl.ds(0, TAIL)],
                            ocnt_hbm.at[c].at[pl.ds(tb, TAIL)])

    return k(ea4, ei3)


def _elu(v):
    return jnp.where(v > 0, v, jnp.exp(jnp.minimum(v, 0.0)) - 1.0)


def _tc_node(batch_ref, x_ref, nsum_ref, cnt_ref):
    f32 = jnp.float32
    bcol = batch_ref[...]
    ids = lax.broadcasted_iota(jnp.int32, (N, B), 1)
    mask = (bcol == ids).astype(f32)
    dn = (((0,), (0,)), ((), ()))
    cnt_ref[...] = lax.dot_general(
        mask, jnp.full((N, 1), 1.0, f32), dn, preferred_element_type=f32)
    nsum_ref[...] = lax.dot_general(
        mask, x_ref[...], dn, preferred_element_type=f32)


def _tc_final(batch8_ref, esum_ref, ecnt_ref, nsum_ref, cnt_ref, u_ref,
              wi_ref, bi_ref, w0_ref, b0_ref, w1_ref, b1_ref,
              wo_ref, bnw_ref, bnb_ref, out_ref):
    f32 = jnp.float32
    dn = (((0,), (0,)), ((), ()))
    cmax = jnp.maximum(cnt_ref[...], 1.0)
    node_mean = nsum_ref[...] / cmax

    es = esum_ref[0] + esum_ref[1]
    c8 = ecnt_ref[0] + ecnt_ref[1]

    ids = lax.broadcasted_iota(jnp.int32, (NP, B), 1)
    egs = jnp.zeros((B, DE), f32)
    for j in range(8):
        w_j = (batch8_ref[:, j:j + 1] == ids).astype(f32) / jnp.maximum(
            c8[:, DE * j:DE * j + 1], 1.0)
        egs = egs + lax.dot_general(w_j, es[:, DE * j:DE * (j + 1)],
                                    dn, preferred_element_type=f32)
    edge_mean = egs / cmax

    wi = wi_ref[...]
    h = (jnp.dot(u_ref[...], wi[0:DU, :], preferred_element_type=f32)
         + jnp.dot(node_mean, wi[DU:DU + DX, :], preferred_element_type=f32)
         + jnp.dot(edge_mean, wi[DU + DX:, :], preferred_element_type=f32)
         + bi_ref[...])
    h = _elu(h)
    skip = h
    h = _elu(jnp.dot(h, w0_ref[...], preferred_element_type=f32) + b0_ref[...])
    h = _elu(jnp.dot(h, w1_ref[...], preferred_element_type=f32) + b1_ref[...])
    h = h + skip
    y = jnp.dot(h, wo_ref[...], preferred_element_type=f32)
    y = y * (bnw_ref[...] * (1.0 / jnp.sqrt(1.0 + 1e-5))) + bnb_ref[...]
    out_ref[...] = _elu(y)


def kernel(x, edge_index, edge_attr, u, batch, W_in, b_in, W_h0, b_h0,
           W_h1, b_h1, W_out, bn_w, bn_b):
    ea4 = edge_attr.T.reshape(2, 8, NCHT, CH).transpose(0, 2, 1, 3)
    ei3 = edge_index.reshape(2, NCHT, CH).transpose(1, 0, 2)
    esum, ecnt = _sc_scatter(ea4, ei3)
    esum_p = esum.reshape(NC, NP, 128)
    ecnt_p = ecnt.reshape(NC, NP, 128)
    batch8 = jnp.pad(batch, (0, NPAD), constant_values=-1).reshape(NP, 8)

    nsum, cnt = pl.pallas_call(
        _tc_node,
        out_shape=[jax.ShapeDtypeStruct((B, DX), jnp.float32),
                   jax.ShapeDtypeStruct((B, 1), jnp.float32)],
    )(batch.reshape(N, 1), x)

    out = pl.pallas_call(
        _tc_final,
        out_shape=jax.ShapeDtypeStruct((B, DO), jnp.float32),
    )(batch8, esum_p, ecnt_p, nsum, cnt, u,
      W_in, b_in.reshape(1, DH), W_h0, b_h0.reshape(1, DH),
      W_h1, b_h1.reshape(1, DH), W_out, bn_w.reshape(1, DO),
      bn_b.reshape(1, DO))
    return out
